# Initial kernel scaffold; baseline (speedup 1.0000x reference)
import jax
import jax.numpy as jnp
from jax.experimental import pallas as pl


def kernel(y_true, y_pred):
    raise NotImplementedError("write your pallas kernel here")


# speedup vs baseline: 12.0160x; 12.0160x over previous
import functools

import jax
import jax.numpy as jnp
from jax import lax
from jax.experimental import pallas as pl
from jax.experimental.pallas import tpu as pltpu
from jax.experimental.pallas import tpu_sc as plsc

_N = 16384
_NSUB = 16
_NW = 32
_EPW = _N // _NW
_NSLICE = 16
_SCOLS = 128

_mesh = plsc.VectorSubcoreMesh(core_axis_name="c", subcore_axis_name="s")
_params = pltpu.CompilerParams(needs_layout_passes=False)


@functools.partial(
    pl.kernel,
    out_type=jax.ShapeDtypeStruct((_NSLICE, _NW, _SCOLS), jnp.float32),
    mesh=_mesh,
    scratch_types=[
        pltpu.VMEM((_EPW,), jnp.int32),
        pltpu.VMEM((_EPW,), jnp.int32),
        pltpu.VMEM((_NSLICE, _SCOLS), jnp.float32),
        pltpu.SemaphoreType.DMA,
    ],
    compiler_params=_params,
)
def _hist_kernel(yt_hbm, yp_hbm, hists_hbm, yt_v, yp_v, hist_v, sem):
    cid = lax.axis_index("c")
    sid = lax.axis_index("s")
    wid = cid * _NSUB + sid
    base = wid * _EPW
    pltpu.sync_copy(yt_hbm.at[pl.ds(base, _EPW)], yt_v)
    pltpu.sync_copy(yp_hbm.at[pl.ds(base, _EPW)], yp_v)

    zeros16 = jnp.zeros((16,), jnp.float32)
    for r in range(_NSLICE):
        for g in range(_SCOLS // 16):
            hist_v[r, pl.ds(g * 16, 16)] = zeros16

    for i in range(_EPW // 16):
        yt = yt_v[pl.ds(i * 16, 16)]
        yp = yp_v[pl.ds(i * 16, 16)]
        hit = jnp.where(yt == yp, 64, 0).astype(jnp.int32)
        row = lax.shift_right_logical(yp, 6)
        col = lax.bitwise_and(yp, 63) + hit
        key = lax.shift_left(row, 7) + col
        cnt, last = plsc.scan_count(key)
        plsc.addupdate_scatter(hist_v, [row, col],
                               cnt.astype(jnp.float32), mask=last)

    copies = [
        pltpu.async_copy(hist_v.at[s], hists_hbm.at[s, wid], sem)
        for s in range(_NSLICE)
    ]
    for c in copies:
        c.wait()


@functools.partial(
    pl.kernel,
    out_type=jax.ShapeDtypeStruct((_NSUB, 32), jnp.float32),
    mesh=_mesh,
    scratch_types=[
        pltpu.VMEM((_NW, _SCOLS), jnp.float32),
        pltpu.VMEM((32,), jnp.float32),
    ],
    compiler_params=_params,
)
def _slice_kernel(hists_hbm, parts_hbm, hv, part_v):
    cid = lax.axis_index("c")
    sid = lax.axis_index("s")

    @pl.when(cid == 0)
    def _():
        pltpu.sync_copy(hists_hbm.at[sid], hv)
        zeros16 = jnp.zeros((16,), jnp.float32)
        sumv = zeros16
        cntv = zeros16
        for g in range(4):
            miss = zeros16
            tp = zeros16
            for t in range(_NW):
                miss = miss + hv[t, pl.ds(g * 16, 16)]
                tp = tp + hv[t, pl.ds(64 + g * 16, 16)]
            tot = miss + tp
            hit = tot > 0.0
            prec = jnp.where(hit, tp / jnp.where(hit, tot, 1.0), 0.0)
            sumv = sumv + prec
            cntv = cntv + jnp.where(hit, 1.0, 0.0)
        part_v[pl.ds(0, 16)] = sumv
        part_v[pl.ds(16, 16)] = cntv
        pltpu.sync_copy(part_v, parts_hbm.at[sid])


@functools.partial(
    pl.kernel,
    out_type=jax.ShapeDtypeStruct((16,), jnp.float32),
    mesh=_mesh,
    scratch_types=[
        pltpu.VMEM((_NSUB, 32), jnp.float32),
        pltpu.VMEM((16,), jnp.float32),
    ],
    compiler_params=_params,
)
def _final_kernel(parts_hbm, out_hbm, loc_v, out_v):
    cid = lax.axis_index("c")
    sid = lax.axis_index("s")

    @pl.when(jnp.logical_and(sid == 0, cid == 0))
    def _():
        pltpu.sync_copy(parts_hbm, loc_v)
        ts = jnp.zeros((16,), jnp.float32)
        tc = jnp.zeros((16,), jnp.float32)
        for t in range(_NSUB):
            ts = ts + loc_v[t, pl.ds(0, 16)]
            tc = tc + loc_v[t, pl.ds(16, 16)]
        total = jnp.sum(ts)
        cnt = jnp.sum(tc)
        out_v[...] = (jnp.full((16,), total, jnp.float32)
                      / jnp.full((16,), cnt, jnp.float32))
        pltpu.sync_copy(out_v, out_hbm)


def kernel(y_true, y_pred):
    hists = _hist_kernel(y_true.astype(jnp.int32), y_pred.astype(jnp.int32))
    parts = _slice_kernel(hists)
    out = _final_kernel(parts)
    return out[0]


# speedup vs baseline: 14.7953x; 1.2313x over previous
import functools

import jax
import jax.numpy as jnp
from jax import lax
from jax.experimental import pallas as pl
from jax.experimental.pallas import tpu as pltpu
from jax.experimental.pallas import tpu_sc as plsc

_N = 16384
_NSUB = 16
_EPT = _N // _NSUB
_NSLICE = 16
_SCOLS = 128

_mesh = plsc.VectorSubcoreMesh(core_axis_name="c", subcore_axis_name="s")
_params = pltpu.CompilerParams(needs_layout_passes=False)


@functools.partial(
    pl.kernel,
    out_type=[
        jax.ShapeDtypeStruct((16,), jnp.float32),
        jax.ShapeDtypeStruct((_NSLICE, _NSUB, _SCOLS), jnp.float32),
        jax.ShapeDtypeStruct((_NSUB, 32), jnp.float32),
    ],
    mesh=_mesh,
    scratch_types=[
        pltpu.VMEM((_EPT,), jnp.int32),
        pltpu.VMEM((_EPT,), jnp.int32),
        pltpu.VMEM((_NSLICE, _SCOLS), jnp.float32),
        pltpu.VMEM((_NSUB, _SCOLS), jnp.float32),
        pltpu.VMEM((32,), jnp.float32),
        pltpu.VMEM((_NSUB, 32), jnp.float32),
        pltpu.VMEM((16,), jnp.float32),
        pltpu.SemaphoreType.DMA,
    ],
    compiler_params=_params,
)
def _prec_kernel(yt_hbm, yp_hbm, out_hbm, exch_hbm, parts_hbm,
                 yt_v, yp_v, hist_v, hv, part_v, loc_v, out_v, sem):
    cid = lax.axis_index("c")
    sid = lax.axis_index("s")

    @pl.when(cid == 0)
    def _():
        base = sid * _EPT
        pltpu.sync_copy(yt_hbm.at[pl.ds(base, _EPT)], yt_v)
        pltpu.sync_copy(yp_hbm.at[pl.ds(base, _EPT)], yp_v)

        zeros16 = jnp.zeros((16,), jnp.float32)
        for r in range(_NSLICE):
            for g in range(_SCOLS // 16):
                hist_v[r, pl.ds(g * 16, 16)] = zeros16

        for i in range(_EPT // 16):
            yt = yt_v[pl.ds(i * 16, 16)]
            yp = yp_v[pl.ds(i * 16, 16)]
            hit = jnp.where(yt == yp, 64, 0).astype(jnp.int32)
            row = lax.shift_right_logical(yp, 6)
            col = lax.bitwise_and(yp, 63) + hit
            key = lax.shift_left(row, 7) + col
            cnt, last = plsc.scan_count(key)
            plsc.addupdate_scatter(hist_v, [row, col],
                                   cnt.astype(jnp.float32), mask=last)

        copies = [
            pltpu.async_copy(hist_v.at[s], exch_hbm.at[s, sid], sem)
            for s in range(_NSLICE)
        ]
        for c in copies:
            c.wait()

    plsc.subcore_barrier()

    @pl.when(cid == 0)
    def _():
        pltpu.sync_copy(exch_hbm.at[sid], hv)
        zeros16 = jnp.zeros((16,), jnp.float32)
        sumv = zeros16
        cntv = zeros16
        for g in range(4):
            miss = zeros16
            tp = zeros16
            for t in range(_NSUB):
                miss = miss + hv[t, pl.ds(g * 16, 16)]
                tp = tp + hv[t, pl.ds(64 + g * 16, 16)]
            tot = miss + tp
            hit = tot > 0.0
            prec = jnp.where(hit, tp / jnp.where(hit, tot, 1.0), 0.0)
            sumv = sumv + prec
            cntv = cntv + jnp.where(hit, 1.0, 0.0)
        part_v[pl.ds(0, 16)] = sumv
        part_v[pl.ds(16, 16)] = cntv
        pltpu.sync_copy(part_v, parts_hbm.at[sid])

    plsc.subcore_barrier()

    @pl.when(jnp.logical_and(sid == 0, cid == 0))
    def _():
        pltpu.sync_copy(parts_hbm, loc_v)
        ts = jnp.zeros((16,), jnp.float32)
        tc = jnp.zeros((16,), jnp.float32)
        for t in range(_NSUB):
            ts = ts + loc_v[t, pl.ds(0, 16)]
            tc = tc + loc_v[t, pl.ds(16, 16)]
        total = jnp.sum(ts)
        cnt = jnp.sum(tc)
        out_v[...] = (jnp.full((16,), total, jnp.float32)
                      / jnp.full((16,), cnt, jnp.float32))
        pltpu.sync_copy(out_v, out_hbm)


def kernel(y_true, y_pred):
    out, _, _ = _prec_kernel(y_true.astype(jnp.int32),
                             y_pred.astype(jnp.int32))
    return out[0]


# speedup vs baseline: 15.7120x; 1.0620x over previous
import functools

import jax
import jax.numpy as jnp
from jax import lax
from jax.experimental import pallas as pl
from jax.experimental.pallas import tpu as pltpu
from jax.experimental.pallas import tpu_sc as plsc

_N = 16384
_NSUB = 16
_EPT = _N // _NSUB
_NSLICE = 16
_SCOLS = 128

_mesh = plsc.VectorSubcoreMesh(core_axis_name="c", subcore_axis_name="s")
_params = pltpu.CompilerParams(needs_layout_passes=False)


@functools.partial(
    pl.kernel,
    out_type=[
        jax.ShapeDtypeStruct((16,), jnp.float32),
        jax.ShapeDtypeStruct((_NSLICE, _NSUB, _SCOLS), jnp.float32),
        jax.ShapeDtypeStruct((_NSUB, 32), jnp.float32),
    ],
    mesh=_mesh,
    scratch_types=[
        pltpu.VMEM((_EPT,), jnp.int32),
        pltpu.VMEM((_EPT,), jnp.int32),
        pltpu.VMEM((_NSLICE, _SCOLS), jnp.float32),
        pltpu.VMEM((_NSUB, _SCOLS), jnp.float32),
        pltpu.VMEM((32,), jnp.float32),
        pltpu.VMEM((_NSUB, 32), jnp.float32),
        pltpu.VMEM((16,), jnp.float32),
        pltpu.SemaphoreType.DMA,
    ],
    compiler_params=_params,
)
def _prec_kernel(yt_hbm, yp_hbm, out_hbm, exch_hbm, parts_hbm,
                 yt_v, yp_v, hist_v, hv, part_v, loc_v, out_v, sem):
    cid = lax.axis_index("c")
    sid = lax.axis_index("s")

    @pl.when(cid == 0)
    def _():
        base = sid * _EPT
        cin = [pltpu.async_copy(yt_hbm.at[pl.ds(base, _EPT)], yt_v, sem),
               pltpu.async_copy(yp_hbm.at[pl.ds(base, _EPT)], yp_v, sem)]

        zeros16 = jnp.zeros((16,), jnp.float32)
        for r in range(_NSLICE):
            for g in range(_SCOLS // 16):
                hist_v[r, pl.ds(g * 16, 16)] = zeros16

        for c in cin:
            c.wait()

        ones16 = jnp.ones((16,), jnp.float32)
        for i in range(_EPT // 16):
            yt = yt_v[pl.ds(i * 16, 16)]
            yp = yp_v[pl.ds(i * 16, 16)]
            hit = jnp.where(yt == yp, 64, 0).astype(jnp.int32)
            row = lax.shift_right_logical(yp, 6)
            col = lax.bitwise_and(yp, 63) + hit
            plsc.addupdate_scatter(hist_v, [row, col], ones16)

        copies = [
            pltpu.async_copy(hist_v.at[s], exch_hbm.at[s, sid], sem)
            for s in range(_NSLICE)
        ]
        for c in copies:
            c.wait()

    plsc.subcore_barrier()

    @pl.when(cid == 0)
    def _():
        pltpu.sync_copy(exch_hbm.at[sid], hv)
        zeros16 = jnp.zeros((16,), jnp.float32)
        sumv = zeros16
        cntv = zeros16
        for g in range(4):
            miss = zeros16
            tp = zeros16
            for t in range(_NSUB):
                miss = miss + hv[t, pl.ds(g * 16, 16)]
                tp = tp + hv[t, pl.ds(64 + g * 16, 16)]
            tot = miss + tp
            hit = tot > 0.0
            prec = jnp.where(hit, tp / jnp.where(hit, tot, 1.0), 0.0)
            sumv = sumv + prec
            cntv = cntv + jnp.where(hit, 1.0, 0.0)
        part_v[pl.ds(0, 16)] = sumv
        part_v[pl.ds(16, 16)] = cntv
        pltpu.sync_copy(part_v, parts_hbm.at[sid])

    plsc.subcore_barrier()

    @pl.when(jnp.logical_and(sid == 0, cid == 0))
    def _():
        pltpu.sync_copy(parts_hbm, loc_v)
        ts = jnp.zeros((16,), jnp.float32)
        tc = jnp.zeros((16,), jnp.float32)
        for t in range(_NSUB):
            ts = ts + loc_v[t, pl.ds(0, 16)]
            tc = tc + loc_v[t, pl.ds(16, 16)]
        total = jnp.sum(ts)
        cnt = jnp.sum(tc)
        out_v[...] = (jnp.full((16,), total, jnp.float32)
                      / jnp.full((16,), cnt, jnp.float32))
        pltpu.sync_copy(out_v, out_hbm)


def kernel(y_true, y_pred):
    out, _, _ = _prec_kernel(y_true.astype(jnp.int32),
                             y_pred.astype(jnp.int32))
    return out[0]


# speedup vs baseline: 15.7238x; 1.0008x over previous
import functools

import jax
import jax.numpy as jnp
from jax import lax
from jax.experimental import pallas as pl
from jax.experimental.pallas import tpu as pltpu
from jax.experimental.pallas import tpu_sc as plsc

_N = 16384
_NSUB = 16
_EPT = _N // _NSUB
_NSLICE = 16
_SCOLS = 128

_mesh = plsc.VectorSubcoreMesh(core_axis_name="c", subcore_axis_name="s")
_params = pltpu.CompilerParams(needs_layout_passes=False)


@functools.partial(
    pl.kernel,
    out_type=[
        jax.ShapeDtypeStruct((16,), jnp.float32),
        jax.ShapeDtypeStruct((_NSLICE, _NSUB, _SCOLS), jnp.float32),
        jax.ShapeDtypeStruct((_NSUB, 32), jnp.float32),
    ],
    mesh=_mesh,
    scratch_types=[
        pltpu.VMEM((_EPT,), jnp.int32),
        pltpu.VMEM((_EPT,), jnp.int32),
        pltpu.VMEM((_NSLICE, _SCOLS), jnp.float32),
        pltpu.VMEM((_NSUB, _SCOLS), jnp.float32),
        pltpu.VMEM((32,), jnp.float32),
        pltpu.VMEM((_NSUB, 32), jnp.float32),
        pltpu.VMEM((16,), jnp.float32),
        pltpu.SemaphoreType.DMA,
    ],
    compiler_params=_params,
)
def _prec_kernel(yt_hbm, yp_hbm, out_hbm, exch_hbm, parts_hbm,
                 yt_v, yp_v, hist_v, hv, part_v, loc_v, out_v, sem):
    cid = lax.axis_index("c")
    sid = lax.axis_index("s")

    @pl.when(cid == 0)
    def _():
        base = sid * _EPT
        cin = [pltpu.async_copy(yt_hbm.at[pl.ds(base, _EPT)], yt_v, sem),
               pltpu.async_copy(yp_hbm.at[pl.ds(base, _EPT)], yp_v, sem)]

        zeros16 = jnp.zeros((16,), jnp.float32)
        for r in range(_NSLICE):
            for g in range(_SCOLS // 16):
                hist_v[r, pl.ds(g * 16, 16)] = zeros16

        for c in cin:
            c.wait()

        ones16 = jnp.ones((16,), jnp.float32)
        for i in range(_EPT // 16):
            yt = yt_v[pl.ds(i * 16, 16)]
            yp = yp_v[pl.ds(i * 16, 16)]
            hit = jnp.where(yt == yp, 64, 0).astype(jnp.int32)
            row = lax.shift_right_logical(yp, 6)
            col = lax.bitwise_and(yp, 63) + hit
            plsc.addupdate_scatter(hist_v, [row, col], ones16)

        copies = [
            pltpu.async_copy(hist_v.at[s], exch_hbm.at[s, sid], sem)
            for s in range(_NSLICE)
        ]
        for c in copies:
            c.wait()

    plsc.subcore_barrier()

    @pl.when(cid == 0)
    def _():
        c1 = pltpu.async_copy(exch_hbm.at[sid, pl.ds(0, 8)],
                              hv.at[pl.ds(0, 8)], sem)
        c2 = pltpu.async_copy(exch_hbm.at[sid, pl.ds(8, 8)],
                              hv.at[pl.ds(8, 8)], sem)
        zeros16 = jnp.zeros((16,), jnp.float32)
        sumv = zeros16
        cntv = zeros16
        c1.wait()
        accs = []
        for g in range(4):
            miss = zeros16
            tp = zeros16
            for t in range(8):
                miss = miss + hv[t, pl.ds(g * 16, 16)]
                tp = tp + hv[t, pl.ds(64 + g * 16, 16)]
            accs.append((miss, tp))
        c2.wait()
        for g in range(4):
            miss, tp = accs[g]
            for t in range(8, _NSUB):
                miss = miss + hv[t, pl.ds(g * 16, 16)]
                tp = tp + hv[t, pl.ds(64 + g * 16, 16)]
            tot = miss + tp
            hit = tot > 0.0
            prec = jnp.where(hit, tp / jnp.where(hit, tot, 1.0), 0.0)
            sumv = sumv + prec
            cntv = cntv + jnp.where(hit, 1.0, 0.0)
        part_v[pl.ds(0, 16)] = sumv
        part_v[pl.ds(16, 16)] = cntv
        pltpu.sync_copy(part_v, parts_hbm.at[sid])

    plsc.subcore_barrier()

    @pl.when(jnp.logical_and(sid == 0, cid == 0))
    def _():
        pltpu.sync_copy(parts_hbm, loc_v)
        ts = jnp.zeros((16,), jnp.float32)
        tc = jnp.zeros((16,), jnp.float32)
        for t in range(_NSUB):
            ts = ts + loc_v[t, pl.ds(0, 16)]
            tc = tc + loc_v[t, pl.ds(16, 16)]
        total = jnp.sum(ts)
        cnt = jnp.sum(tc)
        out_v[...] = (jnp.full((16,), total, jnp.float32)
                      / jnp.full((16,), cnt, jnp.float32))
        pltpu.sync_copy(out_v, out_hbm)


def kernel(y_true, y_pred):
    out, _, _ = _prec_kernel(y_true.astype(jnp.int32),
                             y_pred.astype(jnp.int32))
    return out[0]


# speedup vs baseline: 15.8288x; 1.0067x over previous
import functools

import jax
import jax.numpy as jnp
from jax import lax
from jax.experimental import pallas as pl
from jax.experimental.pallas import tpu as pltpu
from jax.experimental.pallas import tpu_sc as plsc

_N = 16384
_NSUB = 16
_EPT = _N // _NSUB
_NSLICE = 16
_SCOLS = 128

_mesh = plsc.VectorSubcoreMesh(core_axis_name="c", subcore_axis_name="s")
_params = pltpu.CompilerParams(needs_layout_passes=False)


@functools.partial(
    pl.kernel,
    out_type=[
        jax.ShapeDtypeStruct((16,), jnp.float32),
        jax.ShapeDtypeStruct((_NSLICE, _NSUB, _SCOLS), jnp.float32),
        jax.ShapeDtypeStruct((_NSUB, 32), jnp.float32),
    ],
    mesh=_mesh,
    scratch_types=[
        pltpu.VMEM((_EPT,), jnp.int32),
        pltpu.VMEM((_EPT,), jnp.int32),
        pltpu.VMEM((_NSLICE * _SCOLS,), jnp.float32),
        pltpu.VMEM((_NSUB, _SCOLS), jnp.float32),
        pltpu.VMEM((32,), jnp.float32),
        pltpu.VMEM((_NSUB, 32), jnp.float32),
        pltpu.VMEM((16,), jnp.float32),
        pltpu.SemaphoreType.DMA,
    ],
    compiler_params=_params,
)
def _prec_kernel(yt_hbm, yp_hbm, out_hbm, exch_hbm, parts_hbm,
                 yt_v, yp_v, hist_v, hv, part_v, loc_v, out_v, sem):
    cid = lax.axis_index("c")
    sid = lax.axis_index("s")

    @pl.when(cid == 0)
    def _():
        base = sid * _EPT
        cin = [pltpu.async_copy(yt_hbm.at[pl.ds(base, _EPT)], yt_v, sem),
               pltpu.async_copy(yp_hbm.at[pl.ds(base, _EPT)], yp_v, sem)]

        zeros16 = jnp.zeros((16,), jnp.float32)
        for g in range(_NSLICE * _SCOLS // 16):
            hist_v[pl.ds(g * 16, 16)] = zeros16

        for c in cin:
            c.wait()

        ones16 = jnp.ones((16,), jnp.float32)
        for i in range(_EPT // 16):
            yt = yt_v[pl.ds(i * 16, 16)]
            yp = yp_v[pl.ds(i * 16, 16)]
            hit = jnp.where(yt == yp, 64, 0).astype(jnp.int32)
            bin_ = yp + lax.bitwise_and(yp, -64) + hit
            plsc.addupdate_scatter(hist_v, [bin_], ones16)

        copies = [
            pltpu.async_copy(hist_v.at[pl.ds(s * _SCOLS, _SCOLS)],
                             exch_hbm.at[s, sid], sem)
            for s in range(_NSLICE)
        ]
        for c in copies:
            c.wait()

    plsc.subcore_barrier()

    @pl.when(cid == 0)
    def _():
        c1 = pltpu.async_copy(exch_hbm.at[sid, pl.ds(0, 8)],
                              hv.at[pl.ds(0, 8)], sem)
        c2 = pltpu.async_copy(exch_hbm.at[sid, pl.ds(8, 8)],
                              hv.at[pl.ds(8, 8)], sem)
        zeros16 = jnp.zeros((16,), jnp.float32)
        sumv = zeros16
        cntv = zeros16
        c1.wait()
        accs = []
        for g in range(4):
            miss = zeros16
            tp = zeros16
            for t in range(8):
                miss = miss + hv[t, pl.ds(g * 16, 16)]
                tp = tp + hv[t, pl.ds(64 + g * 16, 16)]
            accs.append((miss, tp))
        c2.wait()
        for g in range(4):
            miss, tp = accs[g]
            for t in range(8, _NSUB):
                miss = miss + hv[t, pl.ds(g * 16, 16)]
                tp = tp + hv[t, pl.ds(64 + g * 16, 16)]
            tot = miss + tp
            hit = tot > 0.0
            prec = jnp.where(hit, tp / jnp.where(hit, tot, 1.0), 0.0)
            sumv = sumv + prec
            cntv = cntv + jnp.where(hit, 1.0, 0.0)
        part_v[pl.ds(0, 16)] = sumv
        part_v[pl.ds(16, 16)] = cntv
        pltpu.sync_copy(part_v, parts_hbm.at[sid])

    plsc.subcore_barrier()

    @pl.when(jnp.logical_and(sid == 0, cid == 0))
    def _():
        pltpu.sync_copy(parts_hbm, loc_v)
        ts = jnp.zeros((16,), jnp.float32)
        tc = jnp.zeros((16,), jnp.float32)
        for t in range(_NSUB):
            ts = ts + loc_v[t, pl.ds(0, 16)]
            tc = tc + loc_v[t, pl.ds(16, 16)]
        total = jnp.sum(ts)
        cnt = jnp.sum(tc)
        out_v[...] = (jnp.full((16,), total, jnp.float32)
                      / jnp.full((16,), cnt, jnp.float32))
        pltpu.sync_copy(out_v, out_hbm)


def kernel(y_true, y_pred):
    out, _, _ = _prec_kernel(y_true.astype(jnp.int32),
                             y_pred.astype(jnp.int32))
    return out[0]
